# Initial kernel scaffold; baseline (speedup 1.0000x reference)
import functools

import jax
import jax.numpy as jnp
from jax import lax
from jax.experimental import pallas as pl
from jax.experimental.pallas import tpu as pltpu
from jax.experimental.pallas import tpu_sc as plsc

_EPS = 1e-5

_N = 50000
_K = 16
_PTS_PER_CHUNK = 16
_NCHUNK = _N // _PTS_PER_CHUNK
_NWORKERS = 32
_CHUNKS_PER_W = -(-_NCHUNK // _NWORKERS)
_NB = 512


def _sc_gather_pool(V, F, idxflat, cvec):
    mesh = plsc.VectorSubcoreMesh(core_axis_name="c", subcore_axis_name="s")
    P = _PTS_PER_CHUNK
    R = P * _K

    @functools.partial(
        pl.kernel,
        mesh=mesh,
        out_type=jax.ShapeDtypeStruct((_N, 64), jnp.float32),
        scratch_types=[
            pltpu.VMEM((2, 128), jnp.int32),
            pltpu.VMEM((R, 64), jnp.float32),
            pltpu.VMEM((R, 64), jnp.float32),
            pltpu.VMEM((P, 64), jnp.float32),
            pltpu.VMEM((P, 64), jnp.float32),
            pltpu.VMEM((64,), jnp.float32),
            pltpu.SemaphoreType.DMA,
        ],
    )
    def k(v_hbm, f_hbm, idx_hbm, c_hbm, out_hbm, idxv, nv, nf, ov, og, cv, sem):
        wid = lax.axis_index("s") * 2 + lax.axis_index("c")
        lo = wid * _CHUNKS_PER_W
        hi = jnp.minimum(lo + _CHUNKS_PER_W, _NCHUNK)
        pltpu.sync_copy(c_hbm, cv)

        def chunk_body(c, carry):
            base = c * P
            pltpu.sync_copy(idx_hbm.at[pl.ds(c * R, 128)], idxv.at[0])
            pltpu.sync_copy(idx_hbm.at[pl.ds(c * R + 128, 128)], idxv.at[1])
            hs = [
                pltpu.async_copy(v_hbm.at[idxv.at[0]], nv.at[pl.ds(0, 128)], sem),
                pltpu.async_copy(v_hbm.at[idxv.at[1]], nv.at[pl.ds(128, 128)], sem),
                pltpu.async_copy(f_hbm.at[idxv.at[0]], nf.at[pl.ds(0, 128)], sem),
                pltpu.async_copy(f_hbm.at[idxv.at[1]], nf.at[pl.ds(128, 128)], sem),
                pltpu.async_copy(v_hbm.at[pl.ds(base, P)], ov, sem),
            ]
            for h in hs:
                h.wait()

            def point_body(p, carry2):
                vn0 = ov[p, pl.ds(0, 16)] + cv[pl.ds(0, 16)]
                vn1 = ov[p, pl.ds(16, 16)] + cv[pl.ds(16, 16)]
                vn2 = ov[p, pl.ds(32, 16)] + cv[pl.ds(32, 16)]
                vn3 = ov[p, pl.ds(48, 16)] + cv[pl.ds(48, 16)]
                z = jnp.zeros((16,), jnp.float32)

                def k_body(kk, accs):
                    a0, a1, a2, a3 = accs
                    r = p * _K + kk
                    a0 = a0 + jnp.maximum(vn0 - nv[r, pl.ds(0, 16)], 0.0) * nf[r, pl.ds(0, 16)]
                    a1 = a1 + jnp.maximum(vn1 - nv[r, pl.ds(16, 16)], 0.0) * nf[r, pl.ds(16, 16)]
                    a2 = a2 + jnp.maximum(vn2 - nv[r, pl.ds(32, 16)], 0.0) * nf[r, pl.ds(32, 16)]
                    a3 = a3 + jnp.maximum(vn3 - nv[r, pl.ds(48, 16)], 0.0) * nf[r, pl.ds(48, 16)]
                    return (a0, a1, a2, a3)

                a0, a1, a2, a3 = lax.fori_loop(0, _K, k_body, (z, z, z, z))
                og[p, pl.ds(0, 16)] = a0
                og[p, pl.ds(16, 16)] = a1
                og[p, pl.ds(32, 16)] = a2
                og[p, pl.ds(48, 16)] = a3
                return carry2

            lax.fori_loop(0, P, point_body, 0)
            pltpu.sync_copy(og, out_hbm.at[pl.ds(base, P)])
            return carry

        lax.fori_loop(lo, hi, chunk_body, 0)

    return k(V, F, idxflat, cvec)


def _full(shape):
    return pl.BlockSpec(shape, lambda i: tuple(0 for _ in shape))


def _stage_a(feat2d, xyzT, Wm1, cm1, Wsc, csc, Wa1, Wa2):
    n_blocks = pl.cdiv(_N, _NB)

    def body(feat_ref, xyz_ref, wm1_ref, cm1_ref, wsc_ref, csc_ref,
             wa1_ref, wa2_ref, f1_ref, v1_ref, v2_ref, sco_ref):
        X = feat_ref[...]
        f1 = lax.dot_general(X, wm1_ref[...], (((0,), (1,)), ((), ())),
                             preferred_element_type=jnp.float32)
        f1_ref[...] = jnp.maximum(f1 + cm1_ref[...], 0.0)
        xb = xyz_ref[...]
        wa1 = wa1_ref[...]
        wa2 = wa2_ref[...]
        v1 = (xb[0, :][:, None] * wa1[0, :][None, :]
              + xb[1, :][:, None] * wa1[1, :][None, :]
              + xb[2, :][:, None] * wa1[2, :][None, :])
        v2 = (xb[0, :][:, None] * wa2[0, :][None, :]
              + xb[1, :][:, None] * wa2[1, :][None, :]
              + xb[2, :][:, None] * wa2[2, :][None, :])
        v1_ref[...] = v1
        v2_ref[...] = v2
        sco = lax.dot_general(wsc_ref[...], X, (((1,), (0,)), ((), ())),
                              preferred_element_type=jnp.float32)
        sco_ref[...] = jnp.maximum(sco + csc_ref[...], 0.0)

    return pl.pallas_call(
        body,
        grid=(n_blocks,),
        in_specs=[
            pl.BlockSpec((128, _NB), lambda i: (0, i)),
            pl.BlockSpec((3, _NB), lambda i: (0, i)),
            _full((64, 128)), _full((1, 64)),
            _full((256, 128)), _full((256, 1)),
            _full((3, 64)), _full((3, 64)),
        ],
        out_specs=[
            pl.BlockSpec((_NB, 64), lambda i: (i, 0)),
            pl.BlockSpec((_NB, 64), lambda i: (i, 0)),
            pl.BlockSpec((_NB, 64), lambda i: (i, 0)),
            pl.BlockSpec((256, _NB), lambda i: (0, i)),
        ],
        out_shape=[
            jax.ShapeDtypeStruct((_N, 64), jnp.float32),
            jax.ShapeDtypeStruct((_N, 64), jnp.float32),
            jax.ShapeDtypeStruct((_N, 64), jnp.float32),
            jax.ShapeDtypeStruct((256, _N), jnp.float32),
        ],
    )(feat2d, xyzT, Wm1, cm1, Wsc, csc, Wa1, Wa2)


def _stage_mid(G, W, c):
    n_blocks = pl.cdiv(_N, _NB)

    def body(g_ref, w_ref, c_ref, o_ref):
        y = lax.dot_general(g_ref[...], w_ref[...], (((1,), (1,)), ((), ())),
                            preferred_element_type=jnp.float32)
        o_ref[...] = jnp.maximum(y + c_ref[...], 0.0)

    return pl.pallas_call(
        body,
        grid=(n_blocks,),
        in_specs=[pl.BlockSpec((_NB, 64), lambda i: (i, 0)),
                  _full((64, 64)), _full((1, 64))],
        out_specs=pl.BlockSpec((_NB, 64), lambda i: (i, 0)),
        out_shape=jax.ShapeDtypeStruct((_N, 64), jnp.float32),
    )(G, W, c)


def _stage_out(G2, SCo, Wb2b, cb2b, Wm2, cm2):
    n_blocks = pl.cdiv(_N, _NB)

    def body(g_ref, sco_ref, wb_ref, cb_ref, wm_ref, cm_ref, o_ref):
        f3 = lax.dot_general(g_ref[...], wb_ref[...], (((1,), (1,)), ((), ())),
                             preferred_element_type=jnp.float32)
        f3 = jnp.maximum(f3 + cb_ref[...], 0.0)
        f4 = lax.dot_general(wm_ref[...], f3, (((1,), (1,)), ((), ())),
                             preferred_element_type=jnp.float32)
        f4 = jnp.maximum(f4 + cm_ref[...], 0.0)
        y = f4 + sco_ref[...]
        o_ref[...] = jnp.maximum(y, 0.2 * y)

    return pl.pallas_call(
        body,
        grid=(n_blocks,),
        in_specs=[pl.BlockSpec((_NB, 64), lambda i: (i, 0)),
                  pl.BlockSpec((256, _NB), lambda i: (0, i)),
                  _full((128, 64)), _full((1, 128)),
                  _full((256, 128)), _full((256, 1))],
        out_specs=pl.BlockSpec((256, _NB), lambda i: (0, i)),
        out_shape=jax.ShapeDtypeStruct((256, _N), jnp.float32),
    )(G2, SCo, Wb2b, cb2b, Wm2, cm2)


def kernel(feature, xyz, neigh_idx,
           W_m1, b_m1, g_m1, be_m1,
           W_b1a, b_b1a, g_b1a, be_b1a,
           W_b1b, b_b1b, g_b1b, be_b1b,
           W_b2a, b_b2a, g_b2a, be_b2a,
           W_b2b, b_b2b, g_b2b, be_b2b,
           W_m2, b_m2, g_m2, be_m2,
           W_sc, b_sc, g_sc, be_sc):
    inv = 1.0 / jnp.sqrt(1.0 + _EPS)

    def scale(W, b, g, be):
        s = g * inv
        return W * s[:, None], (b * s + be)

    We_m1, ce_m1 = scale(W_m1, b_m1, g_m1, be_m1)
    We_b1a, ce_b1a = scale(W_b1a, b_b1a, g_b1a, be_b1a)
    We_b1b, ce_b1b = scale(W_b1b, b_b1b, g_b1b, be_b1b)
    We_b2a, ce_b2a = scale(W_b2a, b_b2a, g_b2a, be_b2a)
    We_b2b, ce_b2b = scale(W_b2b, b_b2b, g_b2b, be_b2b)
    We_m2, ce_m2 = scale(W_m2, b_m2, g_m2, be_m2)
    We_sc, ce_sc = scale(W_sc, b_sc, g_sc, be_sc)

    feat2d = feature[0, :, :, 0]
    xyzT = jnp.transpose(xyz[0])
    idxflat = neigh_idx[0].reshape(_N * _K).astype(jnp.int32)

    F1, V1, V2, SCo = _stage_a(
        feat2d, xyzT,
        We_m1, ce_m1.reshape(1, 64),
        We_sc, ce_sc.reshape(256, 1),
        jnp.transpose(We_b1a), jnp.transpose(We_b2a))

    G1 = _sc_gather_pool(V1, F1, idxflat, ce_b1a)
    F2 = _stage_mid(G1, We_b1b * (1.0 / _K), ce_b1b.reshape(1, 64))

    G2 = _sc_gather_pool(V2, F2, idxflat, ce_b2a)
    out = _stage_out(G2, SCo, We_b2b * (1.0 / _K), ce_m2.reshape(256, 1),
                     We_m2, ce_m2.reshape(256, 1))

    return out.reshape(1, 256, _N, 1)


# speedup vs baseline: 19.1170x; 19.1170x over previous
import functools

import jax
import jax.numpy as jnp
from jax import lax
from jax.experimental import pallas as pl
from jax.experimental.pallas import tpu as pltpu
from jax.experimental.pallas import tpu_sc as plsc

_EPS = 1e-5

_N = 50000
_K = 16
_PTS_PER_CHUNK = 16
_NCHUNK = _N // _PTS_PER_CHUNK
_NWORKERS = 32
_CHUNKS_PER_W = -(-_NCHUNK // _NWORKERS)
_NB = 512


def _sc_gather_pool(T, idxflat, cvec):
    mesh = plsc.VectorSubcoreMesh(core_axis_name="c", subcore_axis_name="s")
    P = _PTS_PER_CHUNK
    R = P * _K

    @functools.partial(
        pl.kernel,
        mesh=mesh,
        out_type=jax.ShapeDtypeStruct((_N, 64), jnp.float32),
        scratch_types=[
            pltpu.VMEM((2, 128), jnp.int32),
            pltpu.VMEM((R, 128), jnp.float32),
            pltpu.VMEM((P, 128), jnp.float32),
            pltpu.VMEM((P, 64), jnp.float32),
            pltpu.VMEM((64,), jnp.float32),
            pltpu.SemaphoreType.DMA,
        ],
    )
    def k(t_hbm, idx_hbm, c_hbm, out_hbm, idxv, nt, ot, og, cv, sem):
        wid = lax.axis_index("s") * 2 + lax.axis_index("c")
        lo = wid * _CHUNKS_PER_W
        hi = jnp.minimum(lo + _CHUNKS_PER_W, _NCHUNK)
        pltpu.sync_copy(c_hbm, cv)

        def chunk_body(c, carry):
            base = c * P
            pltpu.sync_copy(idx_hbm.at[pl.ds(c * R, 128)], idxv.at[0])
            pltpu.sync_copy(idx_hbm.at[pl.ds(c * R + 128, 128)], idxv.at[1])
            hs = [
                pltpu.async_copy(t_hbm.at[idxv.at[0]], nt.at[pl.ds(0, 128)], sem),
                pltpu.async_copy(t_hbm.at[idxv.at[1]], nt.at[pl.ds(128, 128)], sem),
                pltpu.async_copy(t_hbm.at[pl.ds(base, P)], ot, sem),
            ]
            for h in hs:
                h.wait()

            def point_body(p, carry2):
                vn0 = ot[p, pl.ds(0, 16)] + cv[pl.ds(0, 16)]
                vn1 = ot[p, pl.ds(16, 16)] + cv[pl.ds(16, 16)]
                vn2 = ot[p, pl.ds(32, 16)] + cv[pl.ds(32, 16)]
                vn3 = ot[p, pl.ds(48, 16)] + cv[pl.ds(48, 16)]
                z = jnp.zeros((16,), jnp.float32)

                def k_body(kk, accs):
                    a0, a1, a2, a3 = accs
                    r = p * _K + kk
                    a0 = a0 + jnp.maximum(vn0 - nt[r, pl.ds(0, 16)], 0.0) * nt[r, pl.ds(64, 16)]
                    a1 = a1 + jnp.maximum(vn1 - nt[r, pl.ds(16, 16)], 0.0) * nt[r, pl.ds(80, 16)]
                    a2 = a2 + jnp.maximum(vn2 - nt[r, pl.ds(32, 16)], 0.0) * nt[r, pl.ds(96, 16)]
                    a3 = a3 + jnp.maximum(vn3 - nt[r, pl.ds(48, 16)], 0.0) * nt[r, pl.ds(112, 16)]
                    return (a0, a1, a2, a3)

                a0, a1, a2, a3 = lax.fori_loop(0, _K, k_body, (z, z, z, z))
                og[p, pl.ds(0, 16)] = a0
                og[p, pl.ds(16, 16)] = a1
                og[p, pl.ds(32, 16)] = a2
                og[p, pl.ds(48, 16)] = a3
                return carry2

            lax.fori_loop(0, P, point_body, 0)
            pltpu.sync_copy(og, out_hbm.at[pl.ds(base, P)])
            return carry

        lax.fori_loop(lo, hi, chunk_body, 0)

    return k(T, idxflat, cvec)


def _full(shape):
    return pl.BlockSpec(shape, lambda i: tuple(0 for _ in shape))


def _stage_a(feat2d, xyzT, Wm1, cm1, Wsc, csc, Wa1, Wa2):
    n_blocks = pl.cdiv(_N, _NB)

    def body(feat_ref, xyz_ref, wm1_ref, cm1_ref, wsc_ref, csc_ref,
             wa1_ref, wa2_ref, t1_ref, v2_ref, sco_ref):
        X = feat_ref[...]
        f1 = lax.dot_general(X, wm1_ref[...], (((0,), (1,)), ((), ())),
                             preferred_element_type=jnp.float32)
        t1_ref[:, 64:128] = jnp.maximum(f1 + cm1_ref[...], 0.0)
        xb = xyz_ref[...]
        wa1 = wa1_ref[...]
        wa2 = wa2_ref[...]
        v1 = (xb[0, :][:, None] * wa1[0, :][None, :]
              + xb[1, :][:, None] * wa1[1, :][None, :]
              + xb[2, :][:, None] * wa1[2, :][None, :])
        v2 = (xb[0, :][:, None] * wa2[0, :][None, :]
              + xb[1, :][:, None] * wa2[1, :][None, :]
              + xb[2, :][:, None] * wa2[2, :][None, :])
        t1_ref[:, 0:64] = v1
        v2_ref[...] = v2
        sco = lax.dot_general(wsc_ref[...], X, (((1,), (0,)), ((), ())),
                              preferred_element_type=jnp.float32)
        sco_ref[...] = jnp.maximum(sco + csc_ref[...], 0.0)

    return pl.pallas_call(
        body,
        grid=(n_blocks,),
        in_specs=[
            pl.BlockSpec((128, _NB), lambda i: (0, i)),
            pl.BlockSpec((3, _NB), lambda i: (0, i)),
            _full((64, 128)), _full((1, 64)),
            _full((256, 128)), _full((256, 1)),
            _full((3, 64)), _full((3, 64)),
        ],
        out_specs=[
            pl.BlockSpec((_NB, 128), lambda i: (i, 0)),
            pl.BlockSpec((_NB, 64), lambda i: (i, 0)),
            pl.BlockSpec((256, _NB), lambda i: (0, i)),
        ],
        out_shape=[
            jax.ShapeDtypeStruct((_N, 128), jnp.float32),
            jax.ShapeDtypeStruct((_N, 64), jnp.float32),
            jax.ShapeDtypeStruct((256, _N), jnp.float32),
        ],
    )(feat2d, xyzT, Wm1, cm1, Wsc, csc, Wa1, Wa2)


def _stage_mid(G, V2, W, c):
    n_blocks = pl.cdiv(_N, _NB)

    def body(g_ref, v2_ref, w_ref, c_ref, t2_ref):
        y = lax.dot_general(g_ref[...], w_ref[...], (((1,), (1,)), ((), ())),
                            preferred_element_type=jnp.float32)
        t2_ref[:, 0:64] = v2_ref[...]
        t2_ref[:, 64:128] = jnp.maximum(y + c_ref[...], 0.0)

    return pl.pallas_call(
        body,
        grid=(n_blocks,),
        in_specs=[pl.BlockSpec((_NB, 64), lambda i: (i, 0)),
                  pl.BlockSpec((_NB, 64), lambda i: (i, 0)),
                  _full((64, 64)), _full((1, 64))],
        out_specs=pl.BlockSpec((_NB, 128), lambda i: (i, 0)),
        out_shape=jax.ShapeDtypeStruct((_N, 128), jnp.float32),
    )(G, V2, W, c)


def _stage_out(G2, SCo, Wb2b, cb2b, Wm2, cm2):
    n_blocks = pl.cdiv(_N, _NB)

    def body(g_ref, sco_ref, wb_ref, cb_ref, wm_ref, cm_ref, o_ref):
        f3 = lax.dot_general(g_ref[...], wb_ref[...], (((1,), (1,)), ((), ())),
                             preferred_element_type=jnp.float32)
        f3 = jnp.maximum(f3 + cb_ref[...], 0.0)
        f4 = lax.dot_general(wm_ref[...], f3, (((1,), (1,)), ((), ())),
                             preferred_element_type=jnp.float32)
        f4 = jnp.maximum(f4 + cm_ref[...], 0.0)
        y = f4 + sco_ref[...]
        o_ref[...] = jnp.maximum(y, 0.2 * y)

    return pl.pallas_call(
        body,
        grid=(n_blocks,),
        in_specs=[pl.BlockSpec((_NB, 64), lambda i: (i, 0)),
                  pl.BlockSpec((256, _NB), lambda i: (0, i)),
                  _full((128, 64)), _full((1, 128)),
                  _full((256, 128)), _full((256, 1))],
        out_specs=pl.BlockSpec((256, _NB), lambda i: (0, i)),
        out_shape=jax.ShapeDtypeStruct((256, _N), jnp.float32),
    )(G2, SCo, Wb2b, cb2b, Wm2, cm2)


def kernel(feature, xyz, neigh_idx,
           W_m1, b_m1, g_m1, be_m1,
           W_b1a, b_b1a, g_b1a, be_b1a,
           W_b1b, b_b1b, g_b1b, be_b1b,
           W_b2a, b_b2a, g_b2a, be_b2a,
           W_b2b, b_b2b, g_b2b, be_b2b,
           W_m2, b_m2, g_m2, be_m2,
           W_sc, b_sc, g_sc, be_sc):
    inv = 1.0 / jnp.sqrt(1.0 + _EPS)

    def scale(W, b, g, be):
        s = g * inv
        return W * s[:, None], (b * s + be)

    We_m1, ce_m1 = scale(W_m1, b_m1, g_m1, be_m1)
    We_b1a, ce_b1a = scale(W_b1a, b_b1a, g_b1a, be_b1a)
    We_b1b, ce_b1b = scale(W_b1b, b_b1b, g_b1b, be_b1b)
    We_b2a, ce_b2a = scale(W_b2a, b_b2a, g_b2a, be_b2a)
    We_b2b, ce_b2b = scale(W_b2b, b_b2b, g_b2b, be_b2b)
    We_m2, ce_m2 = scale(W_m2, b_m2, g_m2, be_m2)
    We_sc, ce_sc = scale(W_sc, b_sc, g_sc, be_sc)

    feat2d = feature[0, :, :, 0]
    xyzT = jnp.transpose(xyz[0])
    idxflat = neigh_idx[0].reshape(_N * _K).astype(jnp.int32)

    T1, V2, SCo = _stage_a(
        feat2d, xyzT,
        We_m1, ce_m1.reshape(1, 64),
        We_sc, ce_sc.reshape(256, 1),
        jnp.transpose(We_b1a), jnp.transpose(We_b2a))

    G1 = _sc_gather_pool(T1, idxflat, ce_b1a)
    T2 = _stage_mid(G1, V2, We_b1b * (1.0 / _K), ce_b1b.reshape(1, 64))

    G2 = _sc_gather_pool(T2, idxflat, ce_b2a)
    out = _stage_out(G2, SCo, We_b2b * (1.0 / _K), ce_b2b.reshape(1, 128),
                     We_m2, ce_m2.reshape(256, 1))

    return out.reshape(1, 256, _N, 1)


# speedup vs baseline: 32.3802x; 1.6938x over previous
import functools

import jax
import jax.numpy as jnp
from jax import lax
from jax.experimental import pallas as pl
from jax.experimental.pallas import tpu as pltpu
from jax.experimental.pallas import tpu_sc as plsc

_EPS = 1e-5

_N = 50000
_K = 16
_PTS_PER_CHUNK = 16
_NCHUNK = _N // _PTS_PER_CHUNK
_NWORKERS = 32
_CHUNKS_PER_W = -(-_NCHUNK // _NWORKERS)
_IDX_ROWS_PER_W = 200
_NB = 512


def _sc_gather_pool(T, idx2d, cvec):
    mesh = plsc.VectorSubcoreMesh(core_axis_name="c", subcore_axis_name="s")
    P = _PTS_PER_CHUNK
    R = P * _K
    CH = _CHUNKS_PER_W

    @functools.partial(
        pl.kernel,
        mesh=mesh,
        out_type=jax.ShapeDtypeStruct((_N, 64), jnp.float32),
        scratch_types=[
            pltpu.VMEM((_IDX_ROWS_PER_W, 128), jnp.int32),
            pltpu.VMEM((R, 128), jnp.float32),
            pltpu.VMEM((R, 128), jnp.float32),
            pltpu.VMEM((P, 128), jnp.float32),
            pltpu.VMEM((P, 128), jnp.float32),
            pltpu.VMEM((P, 64), jnp.float32),
            pltpu.VMEM((P, 64), jnp.float32),
            pltpu.VMEM((64,), jnp.float32),
            pltpu.SemaphoreType.DMA,
            pltpu.SemaphoreType.DMA,
            pltpu.SemaphoreType.DMA,
            pltpu.SemaphoreType.DMA,
        ],
    )
    def k(t_hbm, idx_hbm, c_hbm, out_hbm, idxw, nt0, nt1, ot0, ot1, og0, og1,
          cv, sg0, sg1, so0, so1):
        wid = lax.axis_index("s") * 2 + lax.axis_index("c")
        lo = wid * CH
        nch = jnp.minimum(lo + CH, _NCHUNK) - lo
        pltpu.sync_copy(c_hbm, cv)
        pltpu.sync_copy(idx_hbm.at[pl.ds(wid * _IDX_ROWS_PER_W, _IDX_ROWS_PER_W)], idxw)

        nt = (nt0, nt1)
        ot = (ot0, ot1)
        og = (og0, og1)
        sg = (sg0, sg1)
        so = (so0, so1)

        def issue(i, b):
            li = jnp.minimum(i, nch - 1)
            base = (lo + li) * P
            pltpu.async_copy(t_hbm.at[idxw.at[2 * li]], nt[b].at[pl.ds(0, 128)], sg[b])
            pltpu.async_copy(t_hbm.at[idxw.at[2 * li + 1]], nt[b].at[pl.ds(128, 128)], sg[b])
            pltpu.async_copy(t_hbm.at[pl.ds(base, P)], ot[b], sg[b])

        def wait_gathers(b):
            pltpu.make_async_copy(t_hbm.at[pl.ds(0, 128)], nt[b].at[pl.ds(0, 128)], sg[b]).wait()
            pltpu.make_async_copy(t_hbm.at[pl.ds(0, 128)], nt[b].at[pl.ds(128, 128)], sg[b]).wait()
            pltpu.make_async_copy(t_hbm.at[pl.ds(0, P)], ot[b], sg[b]).wait()

        def wait_out(b):
            pltpu.make_async_copy(out_hbm.at[pl.ds(0, P)], og[b], so[b]).wait()

        issue(0, 0)
        issue(1, 1)

        def outer(ii, carry):
            for b in (0, 1):
                i = 2 * ii + b
                wait_gathers(b)

                @pl.when(ii >= 1)
                def _():
                    wait_out(b)

                ntb, otb, ogb = nt[b], ot[b], og[b]

                def point_body(p, carry2):
                    vn0 = otb[p, pl.ds(0, 16)] + cv[pl.ds(0, 16)]
                    vn1 = otb[p, pl.ds(16, 16)] + cv[pl.ds(16, 16)]
                    vn2 = otb[p, pl.ds(32, 16)] + cv[pl.ds(32, 16)]
                    vn3 = otb[p, pl.ds(48, 16)] + cv[pl.ds(48, 16)]
                    z = jnp.zeros((16,), jnp.float32)
                    a0, a1, a2, a3 = z, z, z, z
                    r0 = p * _K
                    for kk in range(_K):
                        r = r0 + kk
                        a0 = a0 + jnp.maximum(vn0 - ntb[r, pl.ds(0, 16)], 0.0) * ntb[r, pl.ds(64, 16)]
                        a1 = a1 + jnp.maximum(vn1 - ntb[r, pl.ds(16, 16)], 0.0) * ntb[r, pl.ds(80, 16)]
                        a2 = a2 + jnp.maximum(vn2 - ntb[r, pl.ds(32, 16)], 0.0) * ntb[r, pl.ds(96, 16)]
                        a3 = a3 + jnp.maximum(vn3 - ntb[r, pl.ds(48, 16)], 0.0) * ntb[r, pl.ds(112, 16)]
                    ogb[p, pl.ds(0, 16)] = a0
                    ogb[p, pl.ds(16, 16)] = a1
                    ogb[p, pl.ds(32, 16)] = a2
                    ogb[p, pl.ds(48, 16)] = a3
                    return carry2

                lax.fori_loop(0, P, point_body, 0)

                li = jnp.minimum(i, nch - 1)
                pltpu.async_copy(ogb, out_hbm.at[pl.ds((lo + li) * P, P)], so[b])

                @pl.when(ii < (CH // 2) - 1)
                def _():
                    issue(i + 2, b)
            return carry

        lax.fori_loop(0, CH // 2, outer, 0)
        wait_out(0)
        wait_out(1)

    return k(T, idx2d, cvec)


def _full(shape):
    return pl.BlockSpec(shape, lambda i: tuple(0 for _ in shape))


def _stage_a(feat2d, xyzT, Wm1, cm1, Wsc, csc, Wa1, Wa2):
    n_blocks = pl.cdiv(_N, _NB)

    def body(feat_ref, xyz_ref, wm1_ref, cm1_ref, wsc_ref, csc_ref,
             wa1_ref, wa2_ref, t1_ref, v2_ref, sco_ref):
        X = feat_ref[...]
        f1 = lax.dot_general(X, wm1_ref[...], (((0,), (1,)), ((), ())),
                             preferred_element_type=jnp.float32)
        t1_ref[:, 64:128] = jnp.maximum(f1 + cm1_ref[...], 0.0)
        xb = xyz_ref[...]
        wa1 = wa1_ref[...]
        wa2 = wa2_ref[...]
        v1 = (xb[0, :][:, None] * wa1[0, :][None, :]
              + xb[1, :][:, None] * wa1[1, :][None, :]
              + xb[2, :][:, None] * wa1[2, :][None, :])
        v2 = (xb[0, :][:, None] * wa2[0, :][None, :]
              + xb[1, :][:, None] * wa2[1, :][None, :]
              + xb[2, :][:, None] * wa2[2, :][None, :])
        t1_ref[:, 0:64] = v1
        v2_ref[...] = v2
        sco = lax.dot_general(wsc_ref[...], X, (((1,), (0,)), ((), ())),
                              preferred_element_type=jnp.float32)
        sco_ref[...] = jnp.maximum(sco + csc_ref[...], 0.0)

    return pl.pallas_call(
        body,
        grid=(n_blocks,),
        in_specs=[
            pl.BlockSpec((128, _NB), lambda i: (0, i)),
            pl.BlockSpec((3, _NB), lambda i: (0, i)),
            _full((64, 128)), _full((1, 64)),
            _full((256, 128)), _full((256, 1)),
            _full((3, 64)), _full((3, 64)),
        ],
        out_specs=[
            pl.BlockSpec((_NB, 128), lambda i: (i, 0)),
            pl.BlockSpec((_NB, 64), lambda i: (i, 0)),
            pl.BlockSpec((256, _NB), lambda i: (0, i)),
        ],
        out_shape=[
            jax.ShapeDtypeStruct((_N, 128), jnp.float32),
            jax.ShapeDtypeStruct((_N, 64), jnp.float32),
            jax.ShapeDtypeStruct((256, _N), jnp.float32),
        ],
    )(feat2d, xyzT, Wm1, cm1, Wsc, csc, Wa1, Wa2)


def _stage_mid(G, V2, W, c):
    n_blocks = pl.cdiv(_N, _NB)

    def body(g_ref, v2_ref, w_ref, c_ref, t2_ref):
        y = lax.dot_general(g_ref[...], w_ref[...], (((1,), (1,)), ((), ())),
                            preferred_element_type=jnp.float32)
        t2_ref[:, 0:64] = v2_ref[...]
        t2_ref[:, 64:128] = jnp.maximum(y + c_ref[...], 0.0)

    return pl.pallas_call(
        body,
        grid=(n_blocks,),
        in_specs=[pl.BlockSpec((_NB, 64), lambda i: (i, 0)),
                  pl.BlockSpec((_NB, 64), lambda i: (i, 0)),
                  _full((64, 64)), _full((1, 64))],
        out_specs=pl.BlockSpec((_NB, 128), lambda i: (i, 0)),
        out_shape=jax.ShapeDtypeStruct((_N, 128), jnp.float32),
    )(G, V2, W, c)


def _stage_out(G2, SCo, Wb2b, cb2b, Wm2, cm2):
    n_blocks = pl.cdiv(_N, _NB)

    def body(g_ref, sco_ref, wb_ref, cb_ref, wm_ref, cm_ref, o_ref):
        f3 = lax.dot_general(g_ref[...], wb_ref[...], (((1,), (1,)), ((), ())),
                             preferred_element_type=jnp.float32)
        f3 = jnp.maximum(f3 + cb_ref[...], 0.0)
        f4 = lax.dot_general(wm_ref[...], f3, (((1,), (1,)), ((), ())),
                             preferred_element_type=jnp.float32)
        f4 = jnp.maximum(f4 + cm_ref[...], 0.0)
        y = f4 + sco_ref[...]
        o_ref[...] = jnp.maximum(y, 0.2 * y)

    return pl.pallas_call(
        body,
        grid=(n_blocks,),
        in_specs=[pl.BlockSpec((_NB, 64), lambda i: (i, 0)),
                  pl.BlockSpec((256, _NB), lambda i: (0, i)),
                  _full((128, 64)), _full((1, 128)),
                  _full((256, 128)), _full((256, 1))],
        out_specs=pl.BlockSpec((256, _NB), lambda i: (0, i)),
        out_shape=jax.ShapeDtypeStruct((256, _N), jnp.float32),
    )(G2, SCo, Wb2b, cb2b, Wm2, cm2)


def kernel(feature, xyz, neigh_idx,
           W_m1, b_m1, g_m1, be_m1,
           W_b1a, b_b1a, g_b1a, be_b1a,
           W_b1b, b_b1b, g_b1b, be_b1b,
           W_b2a, b_b2a, g_b2a, be_b2a,
           W_b2b, b_b2b, g_b2b, be_b2b,
           W_m2, b_m2, g_m2, be_m2,
           W_sc, b_sc, g_sc, be_sc):
    inv = 1.0 / jnp.sqrt(1.0 + _EPS)

    def scale(W, b, g, be):
        s = g * inv
        return W * s[:, None], (b * s + be)

    We_m1, ce_m1 = scale(W_m1, b_m1, g_m1, be_m1)
    We_b1a, ce_b1a = scale(W_b1a, b_b1a, g_b1a, be_b1a)
    We_b1b, ce_b1b = scale(W_b1b, b_b1b, g_b1b, be_b1b)
    We_b2a, ce_b2a = scale(W_b2a, b_b2a, g_b2a, be_b2a)
    We_b2b, ce_b2b = scale(W_b2b, b_b2b, g_b2b, be_b2b)
    We_m2, ce_m2 = scale(W_m2, b_m2, g_m2, be_m2)
    We_sc, ce_sc = scale(W_sc, b_sc, g_sc, be_sc)

    feat2d = feature[0, :, :, 0]
    xyzT = jnp.transpose(xyz[0])
    idxflat = neigh_idx[0].reshape(_N * _K).astype(jnp.int32)
    nat = 2 * _NWORKERS * _CHUNKS_PER_W
    idx3d = jnp.pad(idxflat, (0, nat * 128 - _N * _K)).reshape(
        _NWORKERS, 2 * _CHUNKS_PER_W, 128)
    idx2d = jnp.pad(idx3d, ((0, 0), (0, _IDX_ROWS_PER_W - 2 * _CHUNKS_PER_W), (0, 0))
                    ).reshape(_NWORKERS * _IDX_ROWS_PER_W, 128)

    T1, V2, SCo = _stage_a(
        feat2d, xyzT,
        We_m1, ce_m1.reshape(1, 64),
        We_sc, ce_sc.reshape(256, 1),
        jnp.transpose(We_b1a), jnp.transpose(We_b2a))

    G1 = _sc_gather_pool(T1, idx2d, ce_b1a)
    T2 = _stage_mid(G1, V2, We_b1b * (1.0 / _K), ce_b1b.reshape(1, 64))

    G2 = _sc_gather_pool(T2, idx2d, ce_b2a)
    out = _stage_out(G2, SCo, We_b2b * (1.0 / _K), ce_b2b.reshape(1, 128),
                     We_m2, ce_m2.reshape(256, 1))

    return out.reshape(1, 256, _N, 1)


# speedup vs baseline: 37.6448x; 1.1626x over previous
import functools

import jax
import jax.numpy as jnp
from jax import lax
from jax.experimental import pallas as pl
from jax.experimental.pallas import tpu as pltpu
from jax.experimental.pallas import tpu_sc as plsc

_EPS = 1e-5

_N = 50000
_K = 16
_PTS_PER_CHUNK = 16
_NCHUNK = _N // _PTS_PER_CHUNK
_NWORKERS = 32
_CHUNKS_PER_W = -(-_NCHUNK // _NWORKERS)
_IDX_ROWS_PER_W = 200
_NB = 2048


def _sc_gather_pool(T, idx2d, cvec):
    mesh = plsc.VectorSubcoreMesh(core_axis_name="c", subcore_axis_name="s")
    P = _PTS_PER_CHUNK
    R = P * _K
    CH = _CHUNKS_PER_W

    @functools.partial(
        pl.kernel,
        mesh=mesh,
        out_type=jax.ShapeDtypeStruct((_N, 64), jnp.float32),
        scratch_types=[
            pltpu.VMEM((_IDX_ROWS_PER_W, 128), jnp.int32),
            pltpu.VMEM((R, 128), jnp.float32),
            pltpu.VMEM((R, 128), jnp.float32),
            pltpu.VMEM((P, 128), jnp.float32),
            pltpu.VMEM((P, 128), jnp.float32),
            pltpu.VMEM((P, 64), jnp.float32),
            pltpu.VMEM((P, 64), jnp.float32),
            pltpu.VMEM((64,), jnp.float32),
            pltpu.SemaphoreType.DMA,
            pltpu.SemaphoreType.DMA,
            pltpu.SemaphoreType.DMA,
            pltpu.SemaphoreType.DMA,
        ],
    )
    def k(t_hbm, idx_hbm, c_hbm, out_hbm, idxw, nt0, nt1, ot0, ot1, og0, og1,
          cv, sg0, sg1, so0, so1):
        wid = lax.axis_index("s") * 2 + lax.axis_index("c")
        lo = wid * CH
        nch = jnp.minimum(lo + CH, _NCHUNK) - lo
        pltpu.sync_copy(c_hbm, cv)
        pltpu.sync_copy(idx_hbm.at[pl.ds(wid * _IDX_ROWS_PER_W, _IDX_ROWS_PER_W)], idxw)

        nt = (nt0, nt1)
        ot = (ot0, ot1)
        og = (og0, og1)
        sg = (sg0, sg1)
        so = (so0, so1)

        def issue(i, b):
            li = jnp.minimum(i, nch - 1)
            base = (lo + li) * P
            pltpu.async_copy(t_hbm.at[idxw.at[2 * li]], nt[b].at[pl.ds(0, 128)], sg[b])
            pltpu.async_copy(t_hbm.at[idxw.at[2 * li + 1]], nt[b].at[pl.ds(128, 128)], sg[b])
            pltpu.async_copy(t_hbm.at[pl.ds(base, P)], ot[b], sg[b])

        def wait_gathers(b):
            pltpu.make_async_copy(t_hbm.at[pl.ds(0, 128)], nt[b].at[pl.ds(0, 128)], sg[b]).wait()
            pltpu.make_async_copy(t_hbm.at[pl.ds(0, 128)], nt[b].at[pl.ds(128, 128)], sg[b]).wait()
            pltpu.make_async_copy(t_hbm.at[pl.ds(0, P)], ot[b], sg[b]).wait()

        def wait_out(b):
            pltpu.make_async_copy(out_hbm.at[pl.ds(0, P)], og[b], so[b]).wait()

        issue(0, 0)
        issue(1, 1)

        def outer(ii, carry):
            for b in (0, 1):
                i = 2 * ii + b
                wait_gathers(b)

                @pl.when(ii >= 1)
                def _():
                    wait_out(b)

                ntb, otb, ogb = nt[b], ot[b], og[b]

                def point_body(p, carry2):
                    vn0 = otb[p, pl.ds(0, 16)] + cv[pl.ds(0, 16)]
                    vn1 = otb[p, pl.ds(16, 16)] + cv[pl.ds(16, 16)]
                    vn2 = otb[p, pl.ds(32, 16)] + cv[pl.ds(32, 16)]
                    vn3 = otb[p, pl.ds(48, 16)] + cv[pl.ds(48, 16)]
                    z = jnp.zeros((16,), jnp.float32)
                    a0, a1, a2, a3 = z, z, z, z
                    r0 = p * _K
                    for kk in range(_K):
                        r = r0 + kk
                        a0 = a0 + jnp.maximum(vn0 - ntb[r, pl.ds(0, 16)], 0.0) * ntb[r, pl.ds(64, 16)]
                        a1 = a1 + jnp.maximum(vn1 - ntb[r, pl.ds(16, 16)], 0.0) * ntb[r, pl.ds(80, 16)]
                        a2 = a2 + jnp.maximum(vn2 - ntb[r, pl.ds(32, 16)], 0.0) * ntb[r, pl.ds(96, 16)]
                        a3 = a3 + jnp.maximum(vn3 - ntb[r, pl.ds(48, 16)], 0.0) * ntb[r, pl.ds(112, 16)]
                    ogb[p, pl.ds(0, 16)] = a0
                    ogb[p, pl.ds(16, 16)] = a1
                    ogb[p, pl.ds(32, 16)] = a2
                    ogb[p, pl.ds(48, 16)] = a3
                    return carry2

                lax.fori_loop(0, P, point_body, 0)

                li = jnp.minimum(i, nch - 1)
                pltpu.async_copy(ogb, out_hbm.at[pl.ds((lo + li) * P, P)], so[b])

                @pl.when(ii < (CH // 2) - 1)
                def _():
                    issue(i + 2, b)
            return carry

        lax.fori_loop(0, CH // 2, outer, 0)
        wait_out(0)
        wait_out(1)

    return k(T, idx2d, cvec)


def _full(shape):
    return pl.BlockSpec(shape, lambda i: tuple(0 for _ in shape))


def _stage_a(feat2d, xyz, Wm1, cm1, Wa1, Wa2):
    n_blocks = pl.cdiv(_N, _NB)

    def body(feat_ref, xyz_ref, wm1_ref, cm1_ref,
             wa1_ref, wa2_ref, t1_ref, v2_ref):
        X = feat_ref[...]
        f1 = lax.dot_general(X, wm1_ref[...], (((0,), (1,)), ((), ())),
                             preferred_element_type=jnp.float32)
        t1_ref[:, 64:128] = jnp.maximum(f1 + cm1_ref[...], 0.0)
        xb = xyz_ref[0]
        wa1 = wa1_ref[...]
        wa2 = wa2_ref[...]
        v1 = (xb[:, 0:1] * wa1[0:1, :]
              + xb[:, 1:2] * wa1[1:2, :]
              + xb[:, 2:3] * wa1[2:3, :])
        v2 = (xb[:, 0:1] * wa2[0:1, :]
              + xb[:, 1:2] * wa2[1:2, :]
              + xb[:, 2:3] * wa2[2:3, :])
        t1_ref[:, 0:64] = v1
        v2_ref[...] = v2

    return pl.pallas_call(
        body,
        grid=(n_blocks,),
        in_specs=[
            pl.BlockSpec((128, _NB), lambda i: (0, i)),
            pl.BlockSpec((1, _NB, 3), lambda i: (0, i, 0)),
            _full((64, 128)), _full((1, 64)),
            _full((3, 64)), _full((3, 64)),
        ],
        out_specs=[
            pl.BlockSpec((_NB, 128), lambda i: (i, 0)),
            pl.BlockSpec((_NB, 64), lambda i: (i, 0)),
        ],
        out_shape=[
            jax.ShapeDtypeStruct((_N, 128), jnp.float32),
            jax.ShapeDtypeStruct((_N, 64), jnp.float32),
        ],
    )(feat2d, xyz, Wm1, cm1, Wa1, Wa2)


def _stage_shortcut(feat2d, Wsc, csc):
    n_blocks = pl.cdiv(_N, _NB)

    def body(feat_ref, wsc_ref, csc_ref, sco_ref):
        X = feat_ref[...]
        sco = lax.dot_general(wsc_ref[...], X, (((1,), (0,)), ((), ())),
                              preferred_element_type=jnp.float32)
        sco_ref[...] = jnp.maximum(sco + csc_ref[...], 0.0)

    return pl.pallas_call(
        body,
        grid=(n_blocks,),
        in_specs=[
            pl.BlockSpec((128, _NB), lambda i: (0, i)),
            _full((256, 128)), _full((256, 1)),
        ],
        out_specs=pl.BlockSpec((256, _NB), lambda i: (0, i)),
        out_shape=jax.ShapeDtypeStruct((256, _N), jnp.float32),
    )(feat2d, Wsc, csc)


def _stage_mid(G, V2, W, c):
    n_blocks = pl.cdiv(_N, _NB)

    def body(g_ref, v2_ref, w_ref, c_ref, t2_ref):
        y = lax.dot_general(g_ref[...], w_ref[...], (((1,), (1,)), ((), ())),
                            preferred_element_type=jnp.float32)
        t2_ref[:, 0:64] = v2_ref[...]
        t2_ref[:, 64:128] = jnp.maximum(y + c_ref[...], 0.0)

    return pl.pallas_call(
        body,
        grid=(n_blocks,),
        in_specs=[pl.BlockSpec((_NB, 64), lambda i: (i, 0)),
                  pl.BlockSpec((_NB, 64), lambda i: (i, 0)),
                  _full((64, 64)), _full((1, 64))],
        out_specs=pl.BlockSpec((_NB, 128), lambda i: (i, 0)),
        out_shape=jax.ShapeDtypeStruct((_N, 128), jnp.float32),
    )(G, V2, W, c)


def _stage_out(G2, SCo, Wb2b, cb2b, Wm2, cm2):
    n_blocks = pl.cdiv(_N, _NB)

    def body(g_ref, sco_ref, wb_ref, cb_ref, wm_ref, cm_ref, o_ref):
        f3 = lax.dot_general(g_ref[...], wb_ref[...], (((1,), (1,)), ((), ())),
                             preferred_element_type=jnp.float32)
        f3 = jnp.maximum(f3 + cb_ref[...], 0.0)
        f4 = lax.dot_general(wm_ref[...], f3, (((1,), (1,)), ((), ())),
                             preferred_element_type=jnp.float32)
        f4 = jnp.maximum(f4 + cm_ref[...], 0.0)
        y = f4 + sco_ref[...]
        o_ref[...] = jnp.maximum(y, 0.2 * y)

    return pl.pallas_call(
        body,
        grid=(n_blocks,),
        in_specs=[pl.BlockSpec((_NB, 64), lambda i: (i, 0)),
                  pl.BlockSpec((256, _NB), lambda i: (0, i)),
                  _full((128, 64)), _full((1, 128)),
                  _full((256, 128)), _full((256, 1))],
        out_specs=pl.BlockSpec((256, _NB), lambda i: (0, i)),
        out_shape=jax.ShapeDtypeStruct((256, _N), jnp.float32),
    )(G2, SCo, Wb2b, cb2b, Wm2, cm2)


def kernel(feature, xyz, neigh_idx,
           W_m1, b_m1, g_m1, be_m1,
           W_b1a, b_b1a, g_b1a, be_b1a,
           W_b1b, b_b1b, g_b1b, be_b1b,
           W_b2a, b_b2a, g_b2a, be_b2a,
           W_b2b, b_b2b, g_b2b, be_b2b,
           W_m2, b_m2, g_m2, be_m2,
           W_sc, b_sc, g_sc, be_sc):
    inv = 1.0 / jnp.sqrt(1.0 + _EPS)

    def scale(W, b, g, be):
        s = g * inv
        return W * s[:, None], (b * s + be)

    We_m1, ce_m1 = scale(W_m1, b_m1, g_m1, be_m1)
    We_b1a, ce_b1a = scale(W_b1a, b_b1a, g_b1a, be_b1a)
    We_b1b, ce_b1b = scale(W_b1b, b_b1b, g_b1b, be_b1b)
    We_b2a, ce_b2a = scale(W_b2a, b_b2a, g_b2a, be_b2a)
    We_b2b, ce_b2b = scale(W_b2b, b_b2b, g_b2b, be_b2b)
    We_m2, ce_m2 = scale(W_m2, b_m2, g_m2, be_m2)
    We_sc, ce_sc = scale(W_sc, b_sc, g_sc, be_sc)

    feat2d = feature[0, :, :, 0]
    idxflat = neigh_idx[0].reshape(_N * _K).astype(jnp.int32)
    nat = 2 * _NWORKERS * _CHUNKS_PER_W
    idx3d = jnp.pad(idxflat, (0, nat * 128 - _N * _K)).reshape(
        _NWORKERS, 2 * _CHUNKS_PER_W, 128)
    idx2d = jnp.pad(idx3d, ((0, 0), (0, _IDX_ROWS_PER_W - 2 * _CHUNKS_PER_W), (0, 0))
                    ).reshape(_NWORKERS * _IDX_ROWS_PER_W, 128)

    T1, V2 = _stage_a(
        feat2d, xyz,
        We_m1, ce_m1.reshape(1, 64),
        jnp.transpose(We_b1a), jnp.transpose(We_b2a))
    SCo = _stage_shortcut(feat2d, We_sc, ce_sc.reshape(256, 1))

    G1 = _sc_gather_pool(T1, idx2d, ce_b1a)
    T2 = _stage_mid(G1, V2, We_b1b * (1.0 / _K), ce_b1b.reshape(1, 64))

    G2 = _sc_gather_pool(T2, idx2d, ce_b2a)
    out = _stage_out(G2, SCo, We_b2b * (1.0 / _K), ce_b2b.reshape(1, 128),
                     We_m2, ce_m2.reshape(256, 1))

    return out.reshape(1, 256, _N, 1)
